# baseline (speedup 1.0000x reference)
import math

import jax
import jax.numpy as jnp
from jax.experimental import pallas as pl
from jax.experimental.pallas import tpu as pltpu

_N = 4096
_T = 0.1
_BD = 128
_C = 256
_BR = 256

_BETA = 2.0 / (_N * _T)
_ZS = sum(math.exp(-_BETA * min(k + 1, _N - 1 - k)) for k in range(_N - 1))
_INV_ZS = 1.0 / _ZS
_NF = float(_N)


def _w_of_k(k):
    kk = jnp.minimum(k + 1.0, _NF - 1.0 - k)
    val = jnp.exp(-_BETA * kk) * _INV_ZS
    return jnp.where((k >= 0.0) & (k <= _NF - 2.0), val, 0.0)


def _diag_threshold_kernel(blk_ref, out_ref, row_sc, col_sc):
    i = pl.program_id(0)
    g = pl.num_programs(0)
    blk = blk_ref[...]
    r_io = jax.lax.broadcasted_iota(jnp.int32, (_BD, _BD), 0)
    c_io = jax.lax.broadcasted_iota(jnp.int32, (_BD, _BD), 1)
    dblk = jnp.where(r_io == c_io, blk, 0.0)
    row_sc[:, pl.ds(i * _BD, _BD)] = jnp.sum(dblk, axis=0, keepdims=True)
    col_sc[pl.ds(i * _BD, _BD), :] = jnp.sum(dblk, axis=1, keepdims=True)

    @pl.when(i == g - 1)
    def _():
        d_row = row_sc[...]

        def body(ci, acc):
            d_col = col_sc[pl.ds(ci * _C, _C), :]
            lt = (d_row < d_col).astype(jnp.float32)
            eq = d_row == d_col
            j_io = jax.lax.broadcasted_iota(jnp.int32, (_C, _N), 1)
            gi = ci * _C + jax.lax.broadcasted_iota(jnp.int32, (_C, _N), 0)
            tie = (eq & (j_io < gi)).astype(jnp.float32)
            rank = jnp.sum(lt + tie, axis=1, keepdims=True)
            cval = 0.5 * (_w_of_k(rank - 1.0) + _w_of_k(rank))
            return acc + jnp.sum(d_col * cval)

        thr = jax.lax.fori_loop(0, _N // _C, body, jnp.float32(0.0))
        out_ref[...] = jnp.where(d_row > thr, 0.0, d_row)


def _diag_embed_kernel(d_ref, out_ref):
    i = pl.program_id(0)
    r_io = jax.lax.broadcasted_iota(jnp.int32, (_BR, _N), 0)
    c_io = jax.lax.broadcasted_iota(jnp.int32, (_BR, _N), 1)
    mask = c_io == r_io + i * _BR
    out_ref[...] = jnp.where(mask, d_ref[...], 0.0)


@jax.jit
def kernel(diagonal_matrix):
    g = _N // _BD
    masked_diag = pl.pallas_call(
        _diag_threshold_kernel,
        grid=(g,),
        in_specs=[pl.BlockSpec((_BD, _BD), lambda i: (i, i))],
        out_specs=pl.BlockSpec((1, _N), lambda i: (0, 0)),
        out_shape=jax.ShapeDtypeStruct((1, _N), jnp.float32),
        scratch_shapes=[
            pltpu.VMEM((1, _N), jnp.float32),
            pltpu.VMEM((_N, 1), jnp.float32),
        ],
        compiler_params=pltpu.CompilerParams(
            dimension_semantics=("arbitrary",),
        ),
    )(diagonal_matrix)

    g2 = _N // _BR
    out = pl.pallas_call(
        _diag_embed_kernel,
        grid=(g2,),
        in_specs=[pl.BlockSpec((1, _N), lambda i: (0, 0))],
        out_specs=pl.BlockSpec((_BR, _N), lambda i: (i, 0)),
        out_shape=jax.ShapeDtypeStruct((_N, _N), jnp.float32),
        compiler_params=pltpu.CompilerParams(
            dimension_semantics=("parallel",),
        ),
    )(masked_diag)
    return out
